# baseline (speedup 1.0000x reference)
import jax
import jax.numpy as jnp
from jax import lax
from jax.experimental import pallas as pl
from jax.experimental.pallas import tpu as pltpu
from jax.experimental.pallas import tpu_sc as plsc

N = 10000
NP = 10240
E = 320000
D = 128
G = 64

NC = 2
NS = 16
NW = NC * NS
EW = E // NW
EWP = 10240
CHUNK = 128
NCHUNK = EWP // CHUNK
RPT = NP // NS
IDXB = 8

BLK = 1024
NBLK = NP // BLK


def _sc_body(src_hbm, dst_hbm, x_hbm, agg_out, degw_out,
             src_v, dst_v, rows0_v, rows1_v, zdeg, ones_v,
             agg_s, degw_s, sem, sem2, sem3, sem4, sem5):
    c = lax.axis_index("c")
    s = lax.axis_index("s")
    w = c * NS + s

    zeros = jnp.zeros((16,), jnp.float32)
    ones = jnp.ones((16,), jnp.float32)

    def zrow(i, carry):
        for k in range(D // 16):
            rows0_v[i, pl.ds(k * 16, 16)] = zeros
        return carry
    lax.fori_loop(0, CHUNK, zrow, 0)

    def zdrow(i, carry):
        zdeg[i, :] = zeros
        ones_v[i, :] = ones
        return carry
    lax.fori_loop(0, CHUNK, zdrow, 0)

    for t in range(RPT // CHUNK):
        pltpu.sync_copy(rows0_v, agg_s.at[pl.ds(s * RPT + t * CHUNK, CHUNK)])
        pltpu.sync_copy(zdeg, degw_s.at[pl.ds(s * RPT + t * CHUNK, CHUNK)])
    plsc.subcore_barrier()

    def outer(t, carry):
        pltpu.sync_copy(src_hbm.at[w, pl.ds(t * IDXB, IDXB)], src_v)
        pltpu.sync_copy(dst_hbm.at[w, pl.ds(t * IDXB, IDXB)], dst_v)

        rows = (rows0_v, rows1_v)
        sems = (sem, sem2)
        ssems = (sem4, sem5)
        gathers = [pltpu.async_copy(x_hbm.at[src_v.at[0]], rows[0], sems[0])]
        scat = [None, None]
        degs = []
        for j in range(IDXB):
            b = j % 2
            if j + 1 < IDXB:
                if scat[1 - b] is not None:
                    scat[1 - b].wait()
                gathers.append(pltpu.async_copy(
                    x_hbm.at[src_v.at[j + 1]], rows[1 - b], sems[1 - b]))
            gathers[j].wait()
            scat[b] = pltpu.async_copy(
                rows[b], agg_s.at[dst_v.at[j]], ssems[b], add=True)
            degs.append(pltpu.async_copy(
                ones_v, degw_s.at[dst_v.at[j]], sem3, add=True))
        for cp in scat:
            if cp is not None:
                cp.wait()
        for dcp in degs:
            dcp.wait()
        return carry
    lax.fori_loop(0, NCHUNK // IDXB, outer, 0)
    plsc.subcore_barrier()

    pltpu.sync_copy(agg_s.at[pl.ds(s * RPT, RPT)],
                    agg_out.at[c, pl.ds(s * RPT, RPT)])
    pltpu.sync_copy(degw_s.at[pl.ds(s * RPT, RPT)],
                    degw_out.at[c, pl.ds(s * RPT, RPT)])


def _sc_call(src3, dst3, x):
    mesh = plsc.VectorSubcoreMesh(core_axis_name="c", subcore_axis_name="s")
    return pl.kernel(
        _sc_body,
        out_type=(jax.ShapeDtypeStruct((NC, NP, D), jnp.float32),
                  jax.ShapeDtypeStruct((NC, NP, 16), jnp.float32)),
        mesh=mesh,
        compiler_params=pltpu.CompilerParams(use_tc_tiling_on_sc=False),
        scratch_types=[
            pltpu.VMEM((IDXB, CHUNK), jnp.int32),
            pltpu.VMEM((IDXB, CHUNK), jnp.int32),
            pltpu.VMEM((CHUNK, D), jnp.float32),
            pltpu.VMEM((CHUNK, D), jnp.float32),
            pltpu.VMEM((CHUNK, 16), jnp.float32),
            pltpu.VMEM((CHUNK, 16), jnp.float32),
            pltpu.VMEM_SHARED((NP, D), jnp.float32),
            pltpu.VMEM_SHARED((NP, 16), jnp.float32),
            pltpu.SemaphoreType.DMA,
            pltpu.SemaphoreType.DMA,
            pltpu.SemaphoreType.DMA,
            pltpu.SemaphoreType.DMA,
            pltpu.SemaphoreType.DMA,
        ],
    )(src3, dst3, x)


def _tc_body(aggp_ref, degp_ref, batch_ref, wg_ref, bg_ref, w1_ref, b1_ref,
             w2_ref, b2_ref, w3_ref, b3_ref, gr_ref, sc_ref,
             pooled_ref, counts_ref):
    i = pl.program_id(0)

    @pl.when(i == 0)
    def _init():
        pooled_ref[...] = jnp.zeros_like(pooled_ref)
        counts_ref[...] = jnp.zeros_like(counts_ref)

    aggsum = aggp_ref[0, :, :] + aggp_ref[1, :, :]
    deg = jnp.sum(degp_ref[0, :, :] + degp_ref[1, :, :], axis=1) * (1.0 / 16.0)
    agg = aggsum * (1.0 / jnp.maximum(deg, 1.0))[:, None]
    h = jnp.maximum(
        jnp.dot(agg, wg_ref[...], precision=lax.Precision.HIGHEST)
        + bg_ref[...], 0.0)

    ids = batch_ref[0, 0, :]
    iota = lax.broadcasted_iota(jnp.int32, (BLK, 128), 1)
    onehot = (ids[:, None] == iota).astype(jnp.float32)
    pooled_ref[...] += lax.dot_general(
        onehot, h, (((0,), (0,)), ((), ())),
        precision=lax.Precision.HIGHEST,
        preferred_element_type=jnp.float32)
    counts_ref[...] += jnp.sum(onehot, axis=0, keepdims=True)

    @pl.when(i == NBLK - 1)
    def _fin():
        cnt = jnp.maximum(counts_ref[0, :], 1.0)
        gr = pooled_ref[...] * (1.0 / cnt)[:, None]
        hi = lax.Precision.HIGHEST
        s1 = jnp.maximum(jnp.dot(gr, w1_ref[...], precision=hi)
                         + b1_ref[...], 0.0)
        s2 = jnp.maximum(jnp.dot(s1, w2_ref[...], precision=hi)
                         + b2_ref[...], 0.0)
        sc = jnp.dot(s2, w3_ref[...], precision=hi) + b3_ref[...]
        gr_ref[...] = gr[:G, :]
        sc_ref[...] = sc[:G, :]


def _tc_call(aggp, degp, batch3, W_gnn, bg2, W1, b12, W2, b22, W3p, b3p):
    wspec = pl.BlockSpec((D, D), lambda i: (0, 0))
    bspec = pl.BlockSpec((1, D), lambda i: (0, 0))
    return pl.pallas_call(
        _tc_body,
        grid=(NBLK,),
        in_specs=[
            pl.BlockSpec((NC, BLK, D), lambda i: (0, i, 0)),
            pl.BlockSpec((NC, BLK, 16), lambda i: (0, i, 0)),
            pl.BlockSpec((1, 1, BLK), lambda i: (i, 0, 0)),
            wspec, bspec, wspec, bspec, wspec, bspec, wspec, bspec,
        ],
        out_specs=[pl.BlockSpec((G, D), lambda i: (0, 0)),
                   pl.BlockSpec((G, D), lambda i: (0, 0))],
        out_shape=[jax.ShapeDtypeStruct((G, D), jnp.float32),
                   jax.ShapeDtypeStruct((G, D), jnp.float32)],
        scratch_shapes=[pltpu.VMEM((128, D), jnp.float32),
                        pltpu.VMEM((1, 128), jnp.float32)],
    )(aggp, degp, batch3, W_gnn, bg2, W1, b12, W2, b22, W3p, b3p)


def kernel(x, edge_index, batch, W_gnn, b_gnn, W1, b1, W2, b2, W3, b3):
    ei2 = edge_index.reshape(2, NW, EW)
    pad = jnp.full((2, NW, EWP - EW), 0, jnp.int32).at[1].set(N)
    ei3 = jnp.concatenate([ei2, pad], axis=2)
    src3 = ei3[0].reshape(NW, NCHUNK, CHUNK)
    dst3 = ei3[1].reshape(NW, NCHUNK, CHUNK)
    aggp, degw = _sc_call(src3, dst3, x)
    batch3 = jnp.concatenate(
        [batch, jnp.full((NP - N,), 127, jnp.int32)]).reshape(NBLK, 1, BLK)
    bg2 = b_gnn.reshape(1, D)
    b12 = b1.reshape(1, D)
    b22 = b2.reshape(1, D)
    W3p = jnp.pad(W3, ((0, 0), (0, D - 1)))
    b3p = jnp.pad(b3, (0, D - 1)).reshape(1, D)
    gr, sc = _tc_call(aggp, degw, batch3, W_gnn, bg2, W1, b12, W2, b22,
                      W3p, b3p)
    return gr, sc[:, :1]


# speedup vs baseline: 2.4125x; 2.4125x over previous
import jax
import jax.numpy as jnp
from jax import lax
from jax.experimental import pallas as pl
from jax.experimental.pallas import tpu as pltpu
from jax.experimental.pallas import tpu_sc as plsc

N = 10000
NP = 10240
E = 320000
D = 128
G = 64

NC = 2
NS = 16
NW = NC * NS
EW = E // NW
CHUNK = 80
NCHUNK = EW // CHUNK
RPT = NP // NS
IDXB = 25

BLK = 1000
NBLK = N // BLK


def _sc_body(src_hbm, dst_hbm, x_hbm, agg_out, degw_out,
             src_v, dst_v, rows0_v, rows1_v, zdeg, ones_v,
             agg_s, degw_s, sem, sem2, sem3, sem4, sem5):
    c = lax.axis_index("c")
    s = lax.axis_index("s")
    w = c * NS + s

    zeros = jnp.zeros((16,), jnp.float32)
    ones = jnp.ones((16,), jnp.float32)

    def zrow(i, carry):
        for k in range(D // 16):
            rows0_v[i, pl.ds(k * 16, 16)] = zeros
        return carry
    lax.fori_loop(0, CHUNK, zrow, 0)

    def zdrow(i, carry):
        zdeg[i, :] = zeros
        ones_v[i, :] = ones
        return carry
    lax.fori_loop(0, CHUNK, zdrow, 0)

    for t in range(RPT // CHUNK):
        pltpu.sync_copy(rows0_v, agg_s.at[pl.ds(s * RPT + t * CHUNK, CHUNK)])
        pltpu.sync_copy(zdeg, degw_s.at[pl.ds(s * RPT + t * CHUNK, CHUNK)])
    plsc.subcore_barrier()

    def outer(t, carry):
        pltpu.sync_copy(src_hbm.at[w, pl.ds(t * IDXB, IDXB)], src_v)
        pltpu.sync_copy(dst_hbm.at[w, pl.ds(t * IDXB, IDXB)], dst_v)

        rows = (rows0_v, rows1_v)
        sems = (sem, sem2)
        ssems = (sem4, sem5)
        gathers = [pltpu.async_copy(x_hbm.at[src_v.at[0]], rows[0], sems[0])]
        scat = [None, None]
        degs = []
        for j in range(IDXB):
            b = j % 2
            if j + 1 < IDXB:
                if scat[1 - b] is not None:
                    scat[1 - b].wait()
                gathers.append(pltpu.async_copy(
                    x_hbm.at[src_v.at[j + 1]], rows[1 - b], sems[1 - b]))
            gathers[j].wait()
            scat[b] = pltpu.async_copy(
                rows[b], agg_s.at[dst_v.at[j]], ssems[b], add=True)
            degs.append(pltpu.async_copy(
                ones_v, degw_s.at[dst_v.at[j]], sem3, add=True))
        for cp in scat:
            if cp is not None:
                cp.wait()
        for dcp in degs:
            dcp.wait()
        return carry
    lax.fori_loop(0, NCHUNK // IDXB, outer, 0)
    plsc.subcore_barrier()

    pltpu.sync_copy(agg_s.at[pl.ds(s * RPT, RPT)],
                    agg_out.at[c, pl.ds(s * RPT, RPT)])
    pltpu.sync_copy(degw_s.at[pl.ds(s * RPT, RPT)],
                    degw_out.at[c, pl.ds(s * RPT, RPT)])


def _sc_call(src3, dst3, x):
    mesh = plsc.VectorSubcoreMesh(core_axis_name="c", subcore_axis_name="s")
    return pl.kernel(
        _sc_body,
        out_type=(jax.ShapeDtypeStruct((NC, NP, D), jnp.float32),
                  jax.ShapeDtypeStruct((NC, NP, 16), jnp.float32)),
        mesh=mesh,
        compiler_params=pltpu.CompilerParams(use_tc_tiling_on_sc=False),
        scratch_types=[
            pltpu.VMEM((IDXB, CHUNK), jnp.int32),
            pltpu.VMEM((IDXB, CHUNK), jnp.int32),
            pltpu.VMEM((CHUNK, D), jnp.float32),
            pltpu.VMEM((CHUNK, D), jnp.float32),
            pltpu.VMEM((CHUNK, 16), jnp.float32),
            pltpu.VMEM((CHUNK, 16), jnp.float32),
            pltpu.VMEM_SHARED((NP, D), jnp.float32),
            pltpu.VMEM_SHARED((NP, 16), jnp.float32),
            pltpu.SemaphoreType.DMA,
            pltpu.SemaphoreType.DMA,
            pltpu.SemaphoreType.DMA,
            pltpu.SemaphoreType.DMA,
            pltpu.SemaphoreType.DMA,
        ],
    )(src3, dst3, x)


def _tc_body(aggp_ref, degp_ref, batch_ref, wg_ref, bg_ref, w1_ref, b1_ref,
             w2_ref, b2_ref, w3_ref, b3_ref, gr_ref, sc_ref,
             pooled_ref, counts_ref):
    i = pl.program_id(0)

    @pl.when(i == 0)
    def _init():
        pooled_ref[...] = jnp.zeros_like(pooled_ref)
        counts_ref[...] = jnp.zeros_like(counts_ref)

    aggsum = aggp_ref[0, :, :] + aggp_ref[1, :, :]
    deg = jnp.sum(degp_ref[0, :, :] + degp_ref[1, :, :], axis=1) * (1.0 / 16.0)
    agg = aggsum * (1.0 / jnp.maximum(deg, 1.0))[:, None]
    h = jnp.maximum(
        jnp.dot(agg, wg_ref[...], precision=lax.Precision.HIGHEST)
        + bg_ref[...], 0.0)

    ids = batch_ref[0, 0, :]
    iota = lax.broadcasted_iota(jnp.int32, (BLK, 128), 1)
    onehot = (ids[:, None] == iota).astype(jnp.float32)
    pooled_ref[...] += lax.dot_general(
        onehot, h, (((0,), (0,)), ((), ())),
        precision=lax.Precision.HIGHEST,
        preferred_element_type=jnp.float32)
    counts_ref[...] += jnp.sum(onehot, axis=0, keepdims=True)

    @pl.when(i == NBLK - 1)
    def _fin():
        cnt = jnp.maximum(counts_ref[0, :], 1.0)
        gr = pooled_ref[...] * (1.0 / cnt)[:, None]
        hi = lax.Precision.HIGHEST
        s1 = jnp.maximum(jnp.dot(gr, w1_ref[...], precision=hi)
                         + b1_ref[...], 0.0)
        s2 = jnp.maximum(jnp.dot(s1, w2_ref[...], precision=hi)
                         + b2_ref[...], 0.0)
        sc = jnp.dot(s2, w3_ref[...], precision=hi) + b3_ref[...]
        gr_ref[...] = gr[:G, :]
        sc_ref[...] = sc[:G, :1]


def _tc_call(aggp, degp, batch3, W_gnn, bg2, W1, b12, W2, b22, W3p, b3p):
    wspec = pl.BlockSpec((D, D), lambda i: (0, 0))
    bspec = pl.BlockSpec((1, D), lambda i: (0, 0))
    return pl.pallas_call(
        _tc_body,
        grid=(NBLK,),
        in_specs=[
            pl.BlockSpec((NC, BLK, D), lambda i: (0, i, 0)),
            pl.BlockSpec((NC, BLK, 16), lambda i: (0, i, 0)),
            pl.BlockSpec((1, 1, BLK), lambda i: (i, 0, 0)),
            wspec, bspec, wspec, bspec, wspec, bspec, wspec, bspec,
        ],
        out_specs=[pl.BlockSpec((G, D), lambda i: (0, 0)),
                   pl.BlockSpec((G, 1), lambda i: (0, 0))],
        out_shape=[jax.ShapeDtypeStruct((G, D), jnp.float32),
                   jax.ShapeDtypeStruct((G, 1), jnp.float32)],
        scratch_shapes=[pltpu.VMEM((128, D), jnp.float32),
                        pltpu.VMEM((1, 128), jnp.float32)],
    )(aggp, degp, batch3, W_gnn, bg2, W1, b12, W2, b22, W3p, b3p)


def kernel(x, edge_index, batch, W_gnn, b_gnn, W1, b1, W2, b2, W3, b3):
    src3 = edge_index[0].reshape(NW, NCHUNK, CHUNK)
    dst3 = edge_index[1].reshape(NW, NCHUNK, CHUNK)
    aggp, degw = _sc_call(src3, dst3, x)
    batch3 = batch.reshape(NBLK, 1, BLK)
    bg2 = b_gnn.reshape(1, D)
    b12 = b1.reshape(1, D)
    b22 = b2.reshape(1, D)
    W3p = jnp.pad(W3, ((0, 0), (0, D - 1)))
    b3p = jnp.pad(b3, (0, D - 1)).reshape(1, D)
    gr, sc = _tc_call(aggp, degw, batch3, W_gnn, bg2, W1, b12, W2, b22,
                      W3p, b3p)
    return gr, sc


# speedup vs baseline: 2.5650x; 1.0632x over previous
import jax
import jax.numpy as jnp
from jax import lax
from jax.experimental import pallas as pl
from jax.experimental.pallas import tpu as pltpu
from jax.experimental.pallas import tpu_sc as plsc

N = 10000
NP = 10240
E = 320000
D = 128
G = 64

NC = 2
NS = 16
NW = NC * NS
EW = E // NW
CHUNK = 80
NCHUNK = EW // CHUNK
RPT = NP // NS
IDXB = 25
DR = NP // 16
DRT = DR // NS

BLK = 1024
NBLK = NP // BLK


def _sc_body(src_hbm, dst_hbm, x_hbm, agg_out, degw_out,
             src_v, dst_v, rows0_v, rows1_v, deg_v, iot_v,
             agg_s, degw_s, sem, sem2, sem3, sem4, sem5):
    c = lax.axis_index("c")
    s = lax.axis_index("s")
    w = c * NS + s

    zeros = jnp.zeros((16,), jnp.float32)
    ones = jnp.ones((16,), jnp.float32)
    iota16 = lax.iota(jnp.int32, 16)

    def zrow(i, carry):
        for k in range(D // 16):
            rows0_v[i, pl.ds(k * 16, 16)] = zeros
        return carry
    lax.fori_loop(0, CHUNK, zrow, 0)

    def zdrow(i, carry):
        deg_v[i, :] = zeros
        return carry
    lax.fori_loop(0, DR, zdrow, 0)

    for r in range(DR // CHUNK):
        for g in range(CHUNK // 16):
            iot_v[r, pl.ds(g * 16, 16)] = iota16 + (r * CHUNK + g * 16)

    for t in range(RPT // CHUNK):
        pltpu.sync_copy(rows0_v, agg_s.at[pl.ds(s * RPT + t * CHUNK, CHUNK)])

    @pl.when(s == 0)
    def _zdeg():
        pltpu.sync_copy(deg_v, degw_s)
    plsc.subcore_barrier()

    def outer(t, carry):
        pltpu.sync_copy(src_hbm.at[w, pl.ds(t * IDXB, IDXB)], src_v)
        pltpu.sync_copy(dst_hbm.at[w, pl.ds(t * IDXB, IDXB)], dst_v)

        rows = (rows0_v, rows1_v)
        sems = (sem, sem2)
        ssems = (sem4, sem5)
        gathers = [pltpu.async_copy(x_hbm.at[src_v.at[0]], rows[0], sems[0])]
        scat = [None, None]
        degs = []
        for j in range(IDXB):
            b = j % 2
            if j + 1 < IDXB:
                if scat[1 - b] is not None:
                    scat[1 - b].wait()
                gathers.append(pltpu.async_copy(
                    x_hbm.at[src_v.at[j + 1]], rows[1 - b], sems[1 - b]))
            gathers[j].wait()
            scat[b] = pltpu.async_copy(
                rows[b], agg_s.at[dst_v.at[j]], ssems[b], add=True)
            for g in range(CHUNK // 16):
                idx16 = dst_v[j, pl.ds(g * 16, 16)]
                plsc.addupdate_scatter(
                    deg_v, [lax.shift_right_logical(idx16, 4), idx16 & 15],
                    ones)
        for cp in scat:
            if cp is not None:
                cp.wait()
        return carry
    lax.fori_loop(0, NCHUNK // IDXB, outer, 0)

    for r in range(DR // CHUNK):
        pltpu.sync_copy(deg_v.at[pl.ds(r * CHUNK, CHUNK)],
                        degw_s.at[iot_v.at[r]], add=True)
    plsc.subcore_barrier()

    pltpu.sync_copy(agg_s.at[pl.ds(s * RPT, RPT)],
                    agg_out.at[c, pl.ds(s * RPT, RPT)])
    pltpu.sync_copy(degw_s.at[pl.ds(s * DRT, DRT)],
                    degw_out.at[c, pl.ds(s * DRT, DRT)])


def _sc_call(src3, dst3, x):
    mesh = plsc.VectorSubcoreMesh(core_axis_name="c", subcore_axis_name="s")
    return pl.kernel(
        _sc_body,
        out_type=(jax.ShapeDtypeStruct((NC, NP, D), jnp.float32),
                  jax.ShapeDtypeStruct((NC, DR, 16), jnp.float32)),
        mesh=mesh,
        compiler_params=pltpu.CompilerParams(use_tc_tiling_on_sc=False, needs_layout_passes=False),
        scratch_types=[
            pltpu.VMEM((IDXB, CHUNK), jnp.int32),
            pltpu.VMEM((IDXB, CHUNK), jnp.int32),
            pltpu.VMEM((CHUNK, D), jnp.float32),
            pltpu.VMEM((CHUNK, D), jnp.float32),
            pltpu.VMEM((DR, 16), jnp.float32),
            pltpu.VMEM((DR // CHUNK, CHUNK), jnp.int32),
            pltpu.VMEM_SHARED((NP, D), jnp.float32),
            pltpu.VMEM_SHARED((DR, 16), jnp.float32),
            pltpu.SemaphoreType.DMA,
            pltpu.SemaphoreType.DMA,
            pltpu.SemaphoreType.DMA,
            pltpu.SemaphoreType.DMA,
            pltpu.SemaphoreType.DMA,
        ],
    )(src3, dst3, x)


def _tc_body(aggp_ref, degp_ref, batch_ref, wg_ref, bg_ref, w1_ref, b1_ref,
             w2_ref, b2_ref, w3_ref, b3_ref, gr_ref, sc_ref,
             pooled_ref, counts_ref):
    i = pl.program_id(0)

    @pl.when(i == 0)
    def _init():
        pooled_ref[...] = jnp.zeros_like(pooled_ref)
        counts_ref[...] = jnp.zeros_like(counts_ref)

    aggsum = aggp_ref[0, :, :] + aggp_ref[1, :, :]
    deg = degp_ref[0, 0, 0, :] + degp_ref[1, 0, 0, :]
    agg = aggsum * (1.0 / jnp.maximum(deg, 1.0))[:, None]
    h = jnp.maximum(
        jnp.dot(agg, wg_ref[...], precision=lax.Precision.HIGHEST)
        + bg_ref[...], 0.0)

    ids = batch_ref[0, 0, :]
    iota = lax.broadcasted_iota(jnp.int32, (BLK, 128), 1)
    onehot = (ids[:, None] == iota).astype(jnp.float32)
    pooled_ref[...] += lax.dot_general(
        onehot, h, (((0,), (0,)), ((), ())),
        precision=lax.Precision.HIGHEST,
        preferred_element_type=jnp.float32)
    counts_ref[...] += jnp.sum(onehot, axis=0, keepdims=True)

    @pl.when(i == NBLK - 1)
    def _fin():
        cnt = jnp.maximum(counts_ref[0, :], 1.0)
        gr = pooled_ref[...] * (1.0 / cnt)[:, None]
        hi = lax.Precision.HIGHEST
        s1 = jnp.maximum(jnp.dot(gr, w1_ref[...], precision=hi)
                         + b1_ref[...], 0.0)
        s2 = jnp.maximum(jnp.dot(s1, w2_ref[...], precision=hi)
                         + b2_ref[...], 0.0)
        sc = jnp.dot(s2, w3_ref[...], precision=hi) + b3_ref[...]
        gr_ref[...] = gr[:G, :]
        sc_ref[...] = sc[:G, :1]


def _tc_call(aggp, degp, batch3, W_gnn, bg2, W1, b12, W2, b22, W3p, b3p):
    wspec = pl.BlockSpec((D, D), lambda i: (0, 0))
    bspec = pl.BlockSpec((1, D), lambda i: (0, 0))
    return pl.pallas_call(
        _tc_body,
        grid=(NBLK,),
        in_specs=[
            pl.BlockSpec((NC, BLK, D), lambda i: (0, i, 0)),
            pl.BlockSpec((NC, 1, 1, BLK), lambda i: (0, i, 0, 0)),
            pl.BlockSpec((1, 1, BLK), lambda i: (i, 0, 0)),
            wspec, bspec, wspec, bspec, wspec, bspec, wspec, bspec,
        ],
        out_specs=[pl.BlockSpec((G, D), lambda i: (0, 0)),
                   pl.BlockSpec((G, 1), lambda i: (0, 0))],
        out_shape=[jax.ShapeDtypeStruct((G, D), jnp.float32),
                   jax.ShapeDtypeStruct((G, 1), jnp.float32)],
        scratch_shapes=[pltpu.VMEM((128, D), jnp.float32),
                        pltpu.VMEM((1, 128), jnp.float32)],
    )(aggp, degp, batch3, W_gnn, bg2, W1, b12, W2, b22, W3p, b3p)


def kernel(x, edge_index, batch, W_gnn, b_gnn, W1, b1, W2, b2, W3, b3):
    src3 = edge_index[0].reshape(NW, NCHUNK, CHUNK)
    dst3 = edge_index[1].reshape(NW, NCHUNK, CHUNK)
    aggp, degw = _sc_call(src3, dst3, x)
    degp = degw.reshape(NC, NBLK, 1, BLK)
    batch3 = jnp.concatenate(
        [batch, jnp.full((NP - N,), 127, jnp.int32)]).reshape(NBLK, 1, BLK)
    bg2 = b_gnn.reshape(1, D)
    b12 = b1.reshape(1, D)
    b22 = b2.reshape(1, D)
    W3p = jnp.pad(W3, ((0, 0), (0, D - 1)))
    b3p = jnp.pad(b3, (0, D - 1)).reshape(1, D)
    gr, sc = _tc_call(aggp, degp, batch3, W_gnn, bg2, W1, b12, W2, b22,
                      W3p, b3p)
    return gr, sc


# speedup vs baseline: 2.8617x; 1.1157x over previous
import jax
import jax.numpy as jnp
from jax import lax
from jax.experimental import pallas as pl
from jax.experimental.pallas import tpu as pltpu
from jax.experimental.pallas import tpu_sc as plsc

N = 10000
NP = 10240
E = 320000
D = 128
G = 64

NC = 2
NS = 16
NW = NC * NS
EW = E // NW
CHUNK = 80
NCHUNK = EW // CHUNK
RPT = NP // NS
IDXB = 25
DR = NP // 16
DRT = DR // NS

BLK = 1024
NBLK = NP // BLK


def _sc_body(src_hbm, dst_hbm, x_hbm, agg_out, degw_out,
             src_v, dst_v, rows0_v, rows1_v, rows2_v, deg_v, iot_v,
             agg_s, degw_s, sem, sem2, sem3, sem4, sem5, sem6):
    c = lax.axis_index("c")
    s = lax.axis_index("s")
    w = c * NS + s

    zeros = jnp.zeros((16,), jnp.float32)
    ones = jnp.ones((16,), jnp.float32)
    iota16 = lax.iota(jnp.int32, 16)

    def zrow(i, carry):
        for k in range(D // 16):
            rows0_v[i, pl.ds(k * 16, 16)] = zeros
        return carry
    lax.fori_loop(0, CHUNK, zrow, 0)

    def zdrow(i, carry):
        deg_v[i, :] = zeros
        return carry
    lax.fori_loop(0, DR, zdrow, 0)

    for r in range(DR // CHUNK):
        for g in range(CHUNK // 16):
            iot_v[r, pl.ds(g * 16, 16)] = iota16 + (r * CHUNK + g * 16)

    for t in range(RPT // CHUNK):
        pltpu.sync_copy(rows0_v, agg_s.at[pl.ds(s * RPT + t * CHUNK, CHUNK)])

    @pl.when(s == 0)
    def _zdeg():
        pltpu.sync_copy(deg_v, degw_s)
    plsc.subcore_barrier()

    def outer(t, carry):
        pltpu.sync_copy(src_hbm.at[w, pl.ds(t * IDXB, IDXB)], src_v)
        pltpu.sync_copy(dst_hbm.at[w, pl.ds(t * IDXB, IDXB)], dst_v)

        rows = (rows0_v, rows1_v, rows2_v)
        sems = (sem, sem2, sem3)
        ssems = (sem4, sem5, sem6)
        gathers = [pltpu.async_copy(x_hbm.at[src_v.at[0]], rows[0], sems[0]),
                   pltpu.async_copy(x_hbm.at[src_v.at[1]], rows[1], sems[1])]
        scat = [None, None, None]
        for j in range(IDXB):
            b = j % 3
            if j + 2 < IDXB:
                nb = (j + 2) % 3
                if scat[nb] is not None:
                    scat[nb].wait()
                gathers.append(pltpu.async_copy(
                    x_hbm.at[src_v.at[j + 2]], rows[nb], sems[nb]))
            gathers[j].wait()
            scat[b] = pltpu.async_copy(
                rows[b], agg_s.at[dst_v.at[j]], ssems[b], add=True)
            for g in range(CHUNK // 16):
                idx16 = dst_v[j, pl.ds(g * 16, 16)]
                plsc.addupdate_scatter(
                    deg_v, [lax.shift_right_logical(idx16, 4), idx16 & 15],
                    ones)
        for cp in scat:
            if cp is not None:
                cp.wait()
        return carry
    lax.fori_loop(0, NCHUNK // IDXB, outer, 0)

    for r in range(DR // CHUNK):
        pltpu.sync_copy(deg_v.at[pl.ds(r * CHUNK, CHUNK)],
                        degw_s.at[iot_v.at[r]], add=True)
    plsc.subcore_barrier()

    pltpu.sync_copy(agg_s.at[pl.ds(s * RPT, RPT)],
                    agg_out.at[c, pl.ds(s * RPT, RPT)])
    pltpu.sync_copy(degw_s.at[pl.ds(s * DRT, DRT)],
                    degw_out.at[c, pl.ds(s * DRT, DRT)])


def _sc_call(src3, dst3, x):
    mesh = plsc.VectorSubcoreMesh(core_axis_name="c", subcore_axis_name="s")
    return pl.kernel(
        _sc_body,
        out_type=(jax.ShapeDtypeStruct((NC, NP, D), jnp.float32),
                  jax.ShapeDtypeStruct((NC, DR, 16), jnp.float32)),
        mesh=mesh,
        compiler_params=pltpu.CompilerParams(use_tc_tiling_on_sc=False, needs_layout_passes=False),
        scratch_types=[
            pltpu.VMEM((IDXB, CHUNK), jnp.int32),
            pltpu.VMEM((IDXB, CHUNK), jnp.int32),
            pltpu.VMEM((CHUNK, D), jnp.float32),
            pltpu.VMEM((CHUNK, D), jnp.float32),
            pltpu.VMEM((CHUNK, D), jnp.float32),
            pltpu.VMEM((DR, 16), jnp.float32),
            pltpu.VMEM((DR // CHUNK, CHUNK), jnp.int32),
            pltpu.VMEM_SHARED((NP, D), jnp.float32),
            pltpu.VMEM_SHARED((DR, 16), jnp.float32),
            pltpu.SemaphoreType.DMA,
            pltpu.SemaphoreType.DMA,
            pltpu.SemaphoreType.DMA,
            pltpu.SemaphoreType.DMA,
            pltpu.SemaphoreType.DMA,
            pltpu.SemaphoreType.DMA,
        ],
    )(src3, dst3, x)


def _tc_body(aggp_ref, degp_ref, batch_ref, wg_ref, bg_ref, w1_ref, b1_ref,
             w2_ref, b2_ref, w3_ref, b3_ref, gr_ref, sc_ref,
             pooled_ref, counts_ref):
    i = pl.program_id(0)

    @pl.when(i == 0)
    def _init():
        pooled_ref[...] = jnp.zeros_like(pooled_ref)
        counts_ref[...] = jnp.zeros_like(counts_ref)

    aggsum = aggp_ref[0, :, :] + aggp_ref[1, :, :]
    deg = degp_ref[0, 0, 0, :] + degp_ref[1, 0, 0, :]
    agg = aggsum * (1.0 / jnp.maximum(deg, 1.0))[:, None]
    h = jnp.maximum(
        jnp.dot(agg, wg_ref[...], precision=lax.Precision.HIGHEST)
        + bg_ref[...], 0.0)

    ids = batch_ref[0, 0, :]
    iota = lax.broadcasted_iota(jnp.int32, (BLK, 128), 1)
    onehot = (ids[:, None] == iota).astype(jnp.float32)
    pooled_ref[...] += lax.dot_general(
        onehot, h, (((0,), (0,)), ((), ())),
        precision=lax.Precision.HIGHEST,
        preferred_element_type=jnp.float32)
    counts_ref[...] += jnp.sum(onehot, axis=0, keepdims=True)

    @pl.when(i == NBLK - 1)
    def _fin():
        cnt = jnp.maximum(counts_ref[0, :], 1.0)
        gr = pooled_ref[...] * (1.0 / cnt)[:, None]
        hi = lax.Precision.HIGHEST
        s1 = jnp.maximum(jnp.dot(gr, w1_ref[...], precision=hi)
                         + b1_ref[...], 0.0)
        s2 = jnp.maximum(jnp.dot(s1, w2_ref[...], precision=hi)
                         + b2_ref[...], 0.0)
        sc = jnp.dot(s2, w3_ref[...], precision=hi) + b3_ref[...]
        gr_ref[...] = gr[:G, :]
        sc_ref[...] = sc[:G, :1]


def _tc_call(aggp, degp, batch3, W_gnn, bg2, W1, b12, W2, b22, W3p, b3p):
    wspec = pl.BlockSpec((D, D), lambda i: (0, 0))
    bspec = pl.BlockSpec((1, D), lambda i: (0, 0))
    return pl.pallas_call(
        _tc_body,
        grid=(NBLK,),
        in_specs=[
            pl.BlockSpec((NC, BLK, D), lambda i: (0, i, 0)),
            pl.BlockSpec((NC, 1, 1, BLK), lambda i: (0, i, 0, 0)),
            pl.BlockSpec((1, 1, BLK), lambda i: (i, 0, 0)),
            wspec, bspec, wspec, bspec, wspec, bspec, wspec, bspec,
        ],
        out_specs=[pl.BlockSpec((G, D), lambda i: (0, 0)),
                   pl.BlockSpec((G, 1), lambda i: (0, 0))],
        out_shape=[jax.ShapeDtypeStruct((G, D), jnp.float32),
                   jax.ShapeDtypeStruct((G, 1), jnp.float32)],
        scratch_shapes=[pltpu.VMEM((128, D), jnp.float32),
                        pltpu.VMEM((1, 128), jnp.float32)],
    )(aggp, degp, batch3, W_gnn, bg2, W1, b12, W2, b22, W3p, b3p)


def kernel(x, edge_index, batch, W_gnn, b_gnn, W1, b1, W2, b2, W3, b3):
    src3 = edge_index[0].reshape(NW, NCHUNK, CHUNK)
    dst3 = edge_index[1].reshape(NW, NCHUNK, CHUNK)
    aggp, degw = _sc_call(src3, dst3, x)
    degp = degw.reshape(NC, NBLK, 1, BLK)
    batch3 = jnp.concatenate(
        [batch, jnp.full((NP - N,), 127, jnp.int32)]).reshape(NBLK, 1, BLK)
    bg2 = b_gnn.reshape(1, D)
    b12 = b1.reshape(1, D)
    b22 = b2.reshape(1, D)
    W3p = jnp.pad(W3, ((0, 0), (0, D - 1)))
    b3p = jnp.pad(b3, (0, D - 1)).reshape(1, D)
    gr, sc = _tc_call(aggp, degp, batch3, W_gnn, bg2, W1, b12, W2, b22,
                      W3p, b3p)
    return gr, sc


# speedup vs baseline: 3.0209x; 1.0556x over previous
import jax
import jax.numpy as jnp
from jax import lax
from jax.experimental import pallas as pl
from jax.experimental.pallas import tpu as pltpu
from jax.experimental.pallas import tpu_sc as plsc

N = 10000
NP = 10240
E = 320000
D = 128
G = 64

NC = 2
NS = 16
NW = NC * NS
EW = E // NW
CHUNK = 80
NCHUNK = EW // CHUNK
RPT = NP // NS
IDXB = 25
DR = NP // 16
DRT = DR // NS

BLK = 1024
NBLK = NP // BLK


def _sc_body(src_hbm, dst_hbm, x_hbm, agg_out, degw_out,
             src_v, dst_v, rows0_v, rows1_v, rows2_v, deg_v, iot_v,
             agg_s, degw_s, sem, sem2, sem3, sem4, sem5, sem6):
    c = lax.axis_index("c")
    s = lax.axis_index("s")
    w = c * NS + s

    zeros = jnp.zeros((16,), jnp.float32)
    ones = jnp.ones((16,), jnp.float32)
    iota16 = lax.iota(jnp.int32, 16)

    def zrow(i, carry):
        for k in range(D // 16):
            rows0_v[i, pl.ds(k * 16, 16)] = zeros
        return carry
    lax.fori_loop(0, CHUNK, zrow, 0)

    def zdrow(i, carry):
        deg_v[i, :] = zeros
        return carry
    lax.fori_loop(0, DR, zdrow, 0)

    for r in range(DR // CHUNK):
        for g in range(CHUNK // 16):
            iot_v[r, pl.ds(g * 16, 16)] = iota16 + (r * CHUNK + g * 16)

    for t in range(RPT // CHUNK):
        pltpu.sync_copy(rows0_v, agg_s.at[pl.ds(s * RPT + t * CHUNK, CHUNK)])

    @pl.when(s == 0)
    def _zdeg():
        pltpu.sync_copy(deg_v, degw_s)
    plsc.subcore_barrier()

    def outer(t, carry):
        pltpu.sync_copy(src_hbm.at[w, pl.ds(t * IDXB, IDXB)], src_v)
        pltpu.sync_copy(dst_hbm.at[w, pl.ds(t * IDXB, IDXB)], dst_v)

        rows = (rows0_v, rows1_v, rows2_v)
        sems = (sem, sem2, sem3)
        ssems = (sem4, sem5, sem6)
        gathers = [pltpu.async_copy(x_hbm.at[src_v.at[0]], rows[0], sems[0]),
                   pltpu.async_copy(x_hbm.at[src_v.at[1]], rows[1], sems[1])]
        scat = [None, None, None]
        for j in range(IDXB):
            b = j % 3
            if j + 2 < IDXB:
                nb = (j + 2) % 3
                if scat[nb] is not None:
                    scat[nb].wait()
                gathers.append(pltpu.async_copy(
                    x_hbm.at[src_v.at[j + 2]], rows[nb], sems[nb]))
            gathers[j].wait()
            scat[b] = pltpu.async_copy(
                rows[b], agg_s.at[dst_v.at[j]], ssems[b], add=True)
            for g in range(CHUNK // 16):
                idx16 = dst_v[j, pl.ds(g * 16, 16)]
                plsc.addupdate_scatter(
                    deg_v, [lax.shift_right_logical(idx16, 4), idx16 & 15],
                    ones)
        for cp in scat:
            if cp is not None:
                cp.wait()
        return carry
    lax.fori_loop(0, NCHUNK // IDXB, outer, 0)

    for r in range(DR // CHUNK):
        pltpu.sync_copy(deg_v.at[pl.ds(r * CHUNK, CHUNK)],
                        degw_s.at[iot_v.at[r]], add=True)
    plsc.subcore_barrier()

    pltpu.sync_copy(agg_s.at[pl.ds(s * RPT, RPT)],
                    agg_out.at[c, pl.ds(s * RPT, RPT)])
    pltpu.sync_copy(degw_s.at[pl.ds(s * DRT, DRT)],
                    degw_out.at[c, pl.ds(s * DRT, DRT)])


def _sc_call(src3, dst3, x):
    mesh = plsc.VectorSubcoreMesh(core_axis_name="c", subcore_axis_name="s")
    return pl.kernel(
        _sc_body,
        out_type=(jax.ShapeDtypeStruct((NC, NP, D), jnp.float32),
                  jax.ShapeDtypeStruct((NC, DR, 16), jnp.float32)),
        mesh=mesh,
        compiler_params=pltpu.CompilerParams(use_tc_tiling_on_sc=False, needs_layout_passes=False),
        scratch_types=[
            pltpu.VMEM((IDXB, CHUNK), jnp.int32),
            pltpu.VMEM((IDXB, CHUNK), jnp.int32),
            pltpu.VMEM((CHUNK, D), jnp.float32),
            pltpu.VMEM((CHUNK, D), jnp.float32),
            pltpu.VMEM((CHUNK, D), jnp.float32),
            pltpu.VMEM((DR, 16), jnp.float32),
            pltpu.VMEM((DR // CHUNK, CHUNK), jnp.int32),
            pltpu.VMEM_SHARED((NP, D), jnp.float32),
            pltpu.VMEM_SHARED((DR, 16), jnp.float32),
            pltpu.SemaphoreType.DMA,
            pltpu.SemaphoreType.DMA,
            pltpu.SemaphoreType.DMA,
            pltpu.SemaphoreType.DMA,
            pltpu.SemaphoreType.DMA,
            pltpu.SemaphoreType.DMA,
        ],
    )(src3, dst3, x)


def _tc_body(aggp_ref, degp_ref, batch_ref, wg_ref, bg_ref, w1_ref, b1_ref,
             w2_ref, b2_ref, w3_ref, b3_ref, gr_ref, sc_ref,
             pooled_ref, counts_ref):
    i = pl.program_id(0)

    @pl.when(i == 0)
    def _init():
        pooled_ref[...] = jnp.zeros_like(pooled_ref)
        counts_ref[...] = jnp.zeros_like(counts_ref)

    aggsum = aggp_ref[0, :, :] + aggp_ref[1, :, :]
    deg = degp_ref[0, 0, 0, :] + degp_ref[1, 0, 0, :]
    agg = aggsum * (1.0 / jnp.maximum(deg, 1.0))[:, None]
    h = jnp.maximum(agg @ wg_ref[...] + bg_ref[...], 0.0)

    ids = batch_ref[0, 0, :]
    iota = lax.broadcasted_iota(jnp.int32, (BLK, 128), 1)
    onehot = (ids[:, None] == iota).astype(jnp.float32)
    pooled_ref[...] += lax.dot_general(
        onehot, h, (((0,), (0,)), ((), ())),
        precision=lax.Precision.HIGHEST,
        preferred_element_type=jnp.float32)
    counts_ref[...] += jnp.sum(onehot, axis=0, keepdims=True)

    @pl.when(i == NBLK - 1)
    def _fin():
        cnt = jnp.maximum(counts_ref[0, :], 1.0)
        gr = pooled_ref[...] * (1.0 / cnt)[:, None]
        s1 = jnp.maximum(gr @ w1_ref[...] + b1_ref[...], 0.0)
        s2 = jnp.maximum(s1 @ w2_ref[...] + b2_ref[...], 0.0)
        sc = s2 @ w3_ref[...] + b3_ref[...]
        gr_ref[...] = gr[:G, :]
        sc_ref[...] = sc[:G, :1]


def _tc_call(aggp, degp, batch3, W_gnn, bg2, W1, b12, W2, b22, W3p, b3p):
    wspec = pl.BlockSpec((D, D), lambda i: (0, 0))
    bspec = pl.BlockSpec((1, D), lambda i: (0, 0))
    return pl.pallas_call(
        _tc_body,
        grid=(NBLK,),
        in_specs=[
            pl.BlockSpec((NC, BLK, D), lambda i: (0, i, 0)),
            pl.BlockSpec((NC, 1, 1, BLK), lambda i: (0, i, 0, 0)),
            pl.BlockSpec((1, 1, BLK), lambda i: (i, 0, 0)),
            wspec, bspec, wspec, bspec, wspec, bspec, wspec, bspec,
        ],
        out_specs=[pl.BlockSpec((G, D), lambda i: (0, 0)),
                   pl.BlockSpec((G, 1), lambda i: (0, 0))],
        out_shape=[jax.ShapeDtypeStruct((G, D), jnp.float32),
                   jax.ShapeDtypeStruct((G, 1), jnp.float32)],
        scratch_shapes=[pltpu.VMEM((128, D), jnp.float32),
                        pltpu.VMEM((1, 128), jnp.float32)],
    )(aggp, degp, batch3, W_gnn, bg2, W1, b12, W2, b22, W3p, b3p)


def kernel(x, edge_index, batch, W_gnn, b_gnn, W1, b1, W2, b2, W3, b3):
    src3 = edge_index[0].reshape(NW, NCHUNK, CHUNK)
    dst3 = edge_index[1].reshape(NW, NCHUNK, CHUNK)
    aggp, degw = _sc_call(src3, dst3, x)
    degp = degw.reshape(NC, NBLK, 1, BLK)
    batch3 = jnp.concatenate(
        [batch, jnp.full((NP - N,), 127, jnp.int32)]).reshape(NBLK, 1, BLK)
    bg2 = b_gnn.reshape(1, D)
    b12 = b1.reshape(1, D)
    b22 = b2.reshape(1, D)
    W3p = jnp.pad(W3, ((0, 0), (0, D - 1)))
    b3p = jnp.pad(b3, (0, D - 1)).reshape(1, D)
    gr, sc = _tc_call(aggp, degp, batch3, W_gnn, bg2, W1, b12, W2, b22,
                      W3p, b3p)
    return gr, sc
